# baseline (speedup 1.0000x reference)
import functools

import jax
import jax.numpy as jnp
from jax import lax
from jax.experimental import pallas as pl
from jax.experimental.pallas import tpu as pltpu
from jax.experimental.pallas import tpu_sc as plsc

N = 10000
NP = 10240
E = 320000
EP = 327680
NT = 32
SHARD = NP // 16
EPT = EP // NT
NG = EPT // 128

IN_DIM = 128
HID_DIM = 256
LAT_DIM = 64

BLK = 2000
GRID = N // BLK

_PREC = lax.Precision.DEFAULT


def _spmm2d(C, ring, chunk, tc_tiling=True):
    mesh = plsc.VectorSubcoreMesh(core_axis_name="c", subcore_axis_name="s")
    nphase = NG // chunk

    @functools.partial(
        pl.kernel,
        out_type=jax.ShapeDtypeStruct((2, NP, C), jnp.float32),
        mesh=mesh,
        compiler_params=pltpu.CompilerParams(use_tc_tiling_on_sc=tc_tiling),
        scratch_types=[
            pltpu.VMEM((chunk, 128), jnp.int32),
            pltpu.VMEM((chunk, 128), jnp.int32),
            [pltpu.VMEM((128, C), jnp.float32)] * ring,
            pltpu.VMEM_SHARED((NP, C), jnp.float32),
            [pltpu.SemaphoreType.DMA] * ring,
            [pltpu.SemaphoreType.DMA] * ring,
        ],
    )
    def spmm(u_hbm, sd_hbm, out_hbm,
             src_ch, dst_ch, rows, acc, gsem, ssem):
        c = lax.axis_index("c")
        s = lax.axis_index("s")
        wid = c * 16 + s
        zero = jnp.zeros((16,), jnp.float32)

        @pl.loop(0, 128)
        def _(i):
            for j in range(C // 16):
                rows[0][i, pl.ds(j * 16, 16)] = zero

        base = s * SHARD
        for i in range(SHARD // 128):
            pltpu.sync_copy(rows[0].at[pl.ds(0, 128)],
                            acc.at[pl.ds(base + i * 128, 128)])
        plsc.subcore_barrier()

        def fire_gather(r, g):
            return pltpu.async_copy(u_hbm.at[src_ch.at[g]], rows[r], gsem[r])

        def fire_scatter(r, g):
            return pltpu.async_copy(rows[r], acc.at[dst_ch.at[g]], ssem[r],
                                    add=True)

        @pl.loop(0, nphase)
        def _(p):
            grow = wid * NG + p * chunk
            pltpu.sync_copy(sd_hbm.at[0, pl.ds(grow, chunk)], src_ch)
            pltpu.sync_copy(sd_hbm.at[3, pl.ds(grow, chunk)], dst_ch)

            for r in range(ring):
                fire_gather(r, r)

            @pl.loop(0, chunk // ring)
            def _(t):
                g0 = t * ring
                for r in range(ring):
                    pltpu.make_async_copy(u_hbm.at[src_ch.at[g0 + r]],
                                          rows[r], gsem[r]).wait()
                    fire_scatter(r, g0 + r)
                for r in range(ring):
                    pltpu.make_async_copy(rows[r], acc.at[dst_ch.at[g0 + r]],
                                          ssem[r]).wait()

                    @pl.when(g0 + ring + r < chunk)
                    def _():
                        fire_gather(r, g0 + ring + r)

        plsc.subcore_barrier()
        pltpu.sync_copy(acc.at[pl.ds(base, SHARD)],
                        out_hbm.at[c, pl.ds(base, SHARD)])

    return spmm


NGC = EP // 16 // 128


def _spmm_cs(ring=8, chunk=80):
    mesh = plsc.VectorSubcoreMesh(core_axis_name="c", subcore_axis_name="s")

    @functools.partial(
        pl.kernel,
        out_type=jax.ShapeDtypeStruct((2, NP, 64), jnp.float32),
        mesh=mesh,
        compiler_params=pltpu.CompilerParams(use_tc_tiling_on_sc=False),
        scratch_types=[
            pltpu.VMEM((chunk, 128), jnp.int32),
            pltpu.VMEM((chunk, 128), jnp.int32),
            [pltpu.VMEM((128, 64), jnp.float32)] * ring,
            pltpu.VMEM_SHARED((NP, 64), jnp.float32),
            [pltpu.SemaphoreType.DMA] * ring,
            [pltpu.SemaphoreType.DMA] * ring,
            pltpu.SemaphoreType.DMA,
        ],
    )
    def spmm(u_hbm, sd_hbm, out_hbm,
             src_all, dst_all, rows, acc, gsem, ssem, isem):
        c = lax.axis_index("c")
        s = lax.axis_index("s")
        icp1 = pltpu.async_copy(sd_hbm.at[1 + c, pl.ds(s * NGC, chunk)],
                                src_all, isem)
        icp2 = pltpu.async_copy(sd_hbm.at[3, pl.ds(s * NGC, chunk)], dst_all,
                                isem)

        zero = jnp.zeros((16,), jnp.float32)

        @pl.loop(0, 128)
        def _(i):
            for j in range(4):
                rows[0][i, pl.ds(j * 16, 16)] = zero

        base = s * SHARD
        for i in range(SHARD // 128):
            pltpu.sync_copy(rows[0].at[pl.ds(0, 128)],
                            acc.at[pl.ds(base + i * 128, 128)])
        plsc.subcore_barrier()
        icp1.wait()
        icp2.wait()

        uh = u_hbm

        def fire_gather(r, g):
            return pltpu.async_copy(uh.at[src_all.at[g]], rows[r], gsem[r])

        def fire_scatter(r, g):
            return pltpu.async_copy(rows[r], acc.at[dst_all.at[g]], ssem[r],
                                    add=True)

        @pl.loop(0, NGC // chunk)
        def _(p):
            @pl.when(p > 0)
            def _():
                icp1b = pltpu.async_copy(
                    sd_hbm.at[1 + c, pl.ds(s * NGC + p * chunk, chunk)],
                    src_all, isem)
                icp2b = pltpu.async_copy(
                    sd_hbm.at[3, pl.ds(s * NGC + p * chunk, chunk)],
                    dst_all, isem)
                icp1b.wait()
                icp2b.wait()

            for r in range(ring):
                fire_gather(r, r)

            @pl.loop(0, chunk // ring)
            def _(t):
                g0 = t * ring
                for r in range(ring):
                    pltpu.make_async_copy(uh.at[src_all.at[g0 + r]],
                                          rows[r], gsem[r]).wait()
                    fire_scatter(r, g0 + r)
                for r in range(ring):
                    pltpu.make_async_copy(rows[r], acc.at[dst_all.at[g0 + r]],
                                          ssem[r]).wait()

                    @pl.when(g0 + ring + r < chunk)
                    def _():
                        fire_gather(r, g0 + ring + r)

        plsc.subcore_barrier()
        pltpu.sync_copy(acc.at[pl.ds(base, SHARD)],
                        out_hbm.at[c, pl.ds(base, SHARD)])

    return spmm


def _degree():
    mesh = plsc.VectorSubcoreMesh(core_axis_name="c", subcore_axis_name="s")
    B = 8

    @functools.partial(
        pl.kernel,
        out_type=jax.ShapeDtypeStruct((2, NP), jnp.float32),
        mesh=mesh,
        compiler_params=pltpu.CompilerParams(use_tc_tiling_on_sc=False),
        scratch_types=[
            pltpu.VMEM((NG, 128), jnp.int32),
            pltpu.VMEM((128,), jnp.float32),
            pltpu.VMEM_SHARED((NP,), jnp.float32),
            pltpu.SemaphoreType.DMA,
            pltpu.SemaphoreType.DMA,
        ],
    )
    def deg(sd_hbm, out_hbm, dst_all, ones_v, acc, ssem, isem):
        c = lax.axis_index("c")
        s = lax.axis_index("s")
        wid = c * 16 + s

        icp = pltpu.async_copy(sd_hbm.at[3, pl.ds(wid * NG, NG)], dst_all,
                               isem)

        one = jnp.full((16,), 1.0, jnp.float32)
        zero = jnp.zeros((16,), jnp.float32)

        base = s * SHARD
        @pl.loop(0, 8)
        def _(i):
            ones_v[pl.ds(i * 16, 16)] = zero

        for i in range(SHARD // 128):
            pltpu.sync_copy(ones_v.at[pl.ds(0, 128)],
                            acc.at[pl.ds(base + i * 128, 128)])

        @pl.loop(0, 8)
        def _(i):
            ones_v[pl.ds(i * 16, 16)] = one

        plsc.subcore_barrier()
        icp.wait()

        @pl.loop(0, NG // B)
        def _(t):
            g0 = t * B
            cps = [
                pltpu.async_copy(ones_v, acc.at[dst_all.at[g0 + r]], ssem,
                                 add=True)
                for r in range(B)
            ]
            for cp in cps:
                cp.wait()

        plsc.subcore_barrier()
        pltpu.sync_copy(acc.at[pl.ds(base, SHARD)],
                        out_hbm.at[c, pl.ds(base, SHARD)])

    return deg


def _tc_pre(dega_ref, degb_ref, x_ref, dinv_ref, u1_ref):
    deg = dega_ref[...] + degb_ref[...] + 1.0
    dv = 1.0 / jnp.sqrt(deg)
    dinv_ref[...] = dv
    u1_ref[...] = x_ref[...] * dv


def _tc_mm1(s_ref, u_ref, dinv_ref, wa_ref, ba_ref, wb_ref, out_ref):
    dv = dinv_ref[...]
    p = dv * (jnp.concatenate([s_ref[0], s_ref[1]], axis=1) + u_ref[...])
    h = jnp.maximum(
        jnp.dot(p, wa_ref[...], precision=_PREC,
                preferred_element_type=jnp.float32) + ba_ref[...], 0.0)
    out_ref[...] = dv * jnp.dot(h, wb_ref[...], precision=_PREC,
                                preferred_element_type=jnp.float32)


def _tc_mm3(s_ref, u_ref, dinv_ref, wa_ref, ba_ref, wb_ref, out_ref):
    dv = dinv_ref[...]
    p = dv * (s_ref[0] + s_ref[1] + u_ref[...])
    h = jnp.maximum(
        jnp.dot(p, wa_ref[...], precision=_PREC,
                preferred_element_type=jnp.float32) + ba_ref[...], 0.0)
    out_ref[...] = dv * jnp.dot(h, wb_ref[...], precision=_PREC,
                                preferred_element_type=jnp.float32)


def _tc_ew(s_ref, u_ref, dinv_ref, b_ref, out_ref):
    dv = dinv_ref[...]
    z = dv * (s_ref[0] + s_ref[1] + u_ref[...]) + b_ref[...]
    out_ref[...] = dv * z


def _tc_final(s_ref, u_ref, dinv_ref, b_ref, out_ref):
    dv = dinv_ref[...]
    out_ref[...] = dv * (jnp.concatenate([s_ref[0], s_ref[1]], axis=1)
                         + u_ref[...]) + b_ref[...]


def _rspec(c):
    return pl.BlockSpec((BLK, c), lambda i: (i, 0))


def _sspec(c):
    return pl.BlockSpec((2, BLK, c), lambda i: (0, i, 0))


def _fspec(shape):
    nd = len(shape)
    return pl.BlockSpec(shape, lambda i: (0,) * nd)


def _uspec():
    return pl.BlockSpec((2, BLK, 64), lambda i: (0, i, 0))


def _call_pre(dega, degb, xp):
    return pl.pallas_call(
        _tc_pre,
        grid=(GRID,),
        in_specs=[_rspec(1), _rspec(1), _rspec(IN_DIM)],
        out_specs=[_rspec(1), _rspec(IN_DIM)],
        out_shape=[
            jax.ShapeDtypeStruct((N, 1), jnp.float32),
            jax.ShapeDtypeStruct((N, IN_DIM), jnp.float32),
        ],
    )(dega, degb, xp)


def _call_mm1(s, u, dinv, wa, ba, wb):
    return pl.pallas_call(
        _tc_mm1,
        grid=(GRID,),
        in_specs=[_uspec(), _rspec(IN_DIM), _rspec(1),
                  _fspec((IN_DIM, HID_DIM)), _fspec((1, HID_DIM)),
                  _fspec((HID_DIM, LAT_DIM))],
        out_specs=_rspec(LAT_DIM),
        out_shape=jax.ShapeDtypeStruct((N, LAT_DIM), jnp.float32),
    )(s, u, dinv, wa, ba, wb)


def _call_mm3(s, u, dinv, wa, ba, wb):
    return pl.pallas_call(
        _tc_mm3,
        grid=(GRID,),
        in_specs=[_sspec(LAT_DIM), _rspec(LAT_DIM), _rspec(1),
                  _fspec((LAT_DIM, HID_DIM)), _fspec((1, HID_DIM)),
                  _fspec((HID_DIM, IN_DIM))],
        out_specs=_rspec(IN_DIM),
        out_shape=jax.ShapeDtypeStruct((N, IN_DIM), jnp.float32),
    )(s, u, dinv, wa, ba, wb)


def _call_ew(s, u, dinv, b, c):
    return pl.pallas_call(
        _tc_ew,
        grid=(GRID,),
        in_specs=[_sspec(c), _rspec(c), _rspec(1), _fspec((1, c))],
        out_specs=_rspec(c),
        out_shape=jax.ShapeDtypeStruct((N, c), jnp.float32),
    )(s, u, dinv, b)


def _call_final(s, u, dinv, b):
    return pl.pallas_call(
        _tc_final,
        grid=(GRID,),
        in_specs=[_uspec(), _rspec(IN_DIM), _rspec(1),
                  _fspec((1, IN_DIM))],
        out_specs=_rspec(IN_DIM),
        out_shape=jax.ShapeDtypeStruct((N, IN_DIM), jnp.float32),
    )(s, u, dinv, b)


def kernel(x, edge_index, W1, b1, W2, b2, W3, b3, W4, b4):
    npad = EP - E
    pidx = lax.iota(jnp.int32, npad)
    pads = jnp.stack([pidx % N, N + pidx % (NP - N)])
    ei = jnp.concatenate([edge_index.astype(jnp.int32), pads], axis=1)
    sfull, dfull = ei[0], ei[1]
    sd = jnp.stack([sfull, 2 * sfull, 2 * sfull + 1,
                    dfull]).reshape(4, EP // 128, 128)

    spmm128 = _spmm_cs(ring=8, chunk=80)
    spmm64 = _spmm2d(64, ring=8, chunk=80, tc_tiling=False)

    deg2 = _degree()(sd)
    dinv, u1 = _call_pre(deg2[0].reshape(NP, 1)[:N], deg2[1].reshape(NP, 1)[:N],
                         x)

    s1 = spmm128(u1.reshape(2 * N, 64), sd)
    u2 = _call_mm1(s1, u1, dinv, W1, b1.reshape(1, -1), W2)

    s2 = spmm64(u2, sd)
    u3 = _call_ew(s2, u2, dinv, b2.reshape(1, -1), LAT_DIM)

    s3 = spmm64(u3, sd)
    u4 = _call_mm3(s3, u3, dinv, W3, b3.reshape(1, -1), W4)

    s4 = spmm128(u4.reshape(2 * N, 64), sd)
    return _call_final(s4, u4, dinv, b4.reshape(1, -1))
